# baseline (speedup 1.0000x reference)
import functools

import jax
import jax.numpy as jnp
from jax import lax
from jax.experimental import pallas as pl
from jax.experimental.pallas import tpu as pltpu
from jax.experimental.pallas import tpu_sc as plsc

B = 4
N = 50000
E = 1600000
CBG = 30000
D_IN = 8
D_HID = 16
D_OUT = 8

NC, NS, L = 2, 16, 16
NW = NC * NS
N_VECS = N // L

DEG_CH = 4000
DEG_EDGES = E // 8
AGG_CH = 4000

_SC_MESH = plsc.VectorSubcoreMesh(
    core_axis_name="c", subcore_axis_name="s", num_cores=NC, num_subcores=NS)
_SC_PARAMS = pltpu.CompilerParams(needs_layout_passes=False)


def _zero_fill(ref):
    zeros = jnp.zeros((L,), jnp.float32)

    @plsc.parallel_loop(0, N_VECS, unroll=8)
    def body(i):
        ref[pl.ds(i * L, L)] = zeros


@functools.partial(
    pl.kernel, mesh=_SC_MESH, compiler_params=_SC_PARAMS,
    out_type=(jax.ShapeDtypeStruct((NW * N,), jnp.float32),
              jax.ShapeDtypeStruct((NW * N,), jnp.float32)),
    scratch_types=[
        pltpu.VMEM((N,), jnp.float32),
        pltpu.VMEM((N,), jnp.float32),
        pltpu.VMEM((DEG_CH,), jnp.int32),
        pltpu.VMEM((DEG_CH,), jnp.int32),
        pltpu.VMEM((DEG_CH,), jnp.int32),
        pltpu.VMEM((DEG_CH,), jnp.int32),
        pltpu.SemaphoreType.DMA,
        pltpu.SemaphoreType.DMA,
    ],
)
def _sc_degrees(src_hbm, dst_hbm, cnt_src_hbm, cnt_dst_hbm,
                cs_v, cd_v, src_b0, src_b1, dst_b0, dst_b1, sem0, sem1):
    wid = lax.axis_index("s") * NC + lax.axis_index("c")
    b = wid // 8
    base = b * E + (wid % 8) * DEG_EDGES
    n_chunks = DEG_EDGES // DEG_CH
    slots = ((src_b0, dst_b0, sem0), (src_b1, dst_b1, sem1))

    def issue(c, k):
        s_b, d_b, sem = slots[k]
        off = base + c * DEG_CH
        pltpu.async_copy(src_hbm.at[pl.ds(off, DEG_CH)], s_b, sem)
        pltpu.async_copy(dst_hbm.at[pl.ds(off, DEG_CH)], d_b, sem)

    def drain(k):
        s_b, d_b, sem = slots[k]
        pltpu.make_async_copy(src_hbm.at[pl.ds(0, DEG_CH)], s_b, sem).wait()
        pltpu.make_async_copy(dst_hbm.at[pl.ds(0, DEG_CH)], d_b, sem).wait()

    issue(0, 0)
    issue(1, 1)
    _zero_fill(cs_v)
    _zero_fill(cd_v)
    ones = jnp.ones((L,), jnp.float32)

    def pair(g, _):
        for k in range(2):
            c = g * 2 + k
            s_b, d_b, _sem = slots[k]
            drain(k)

            @plsc.parallel_loop(0, DEG_CH // L, unroll=8)
            def vec(i, s_b=s_b, d_b=d_b):
                o = i * L
                plsc.addupdate_scatter(cs_v, [s_b[pl.ds(o, L)]], ones)
                plsc.addupdate_scatter(cd_v, [d_b[pl.ds(o, L)]], ones)

            @pl.when(c + 2 < n_chunks)
            def _(c=c, k=k):
                issue(c + 2, k)
        return 0

    lax.fori_loop(0, n_chunks // 2, pair, 0)
    pltpu.sync_copy(cs_v, cnt_src_hbm.at[pl.ds(wid * N, N)])
    pltpu.sync_copy(cd_v, cnt_dst_hbm.at[pl.ds(wid * N, N)])


@functools.partial(
    pl.kernel, mesh=_SC_MESH, compiler_params=_SC_PARAMS,
    out_type=jax.ShapeDtypeStruct((NW * N,), jnp.float32),
    scratch_types=[
        pltpu.VMEM((N,), jnp.float32),
        pltpu.VMEM((N,), jnp.float32),
        pltpu.VMEM((AGG_CH,), jnp.int32),
        pltpu.VMEM((AGG_CH,), jnp.int32),
        pltpu.VMEM((AGG_CH,), jnp.int32),
        pltpu.VMEM((AGG_CH,), jnp.int32),
        pltpu.VMEM((AGG_CH,), jnp.float32),
        pltpu.VMEM((AGG_CH,), jnp.float32),
        pltpu.SemaphoreType.DMA,
        pltpu.SemaphoreType.DMA,
    ],
)
def _sc_aggregate(src_hbm, dst_hbm, ew_hbm, tab_hbm, agg_hbm,
                  tab_v, acc_v, src_b0, src_b1, dst_b0, dst_b1,
                  ew_b0, ew_b1, sem0, sem1):
    wid = lax.axis_index("s") * NC + lax.axis_index("c")
    b = wid // 8
    n_chunks = E // AGG_CH
    slots = ((src_b0, dst_b0, ew_b0, sem0), (src_b1, dst_b1, ew_b1, sem1))

    def issue(c, k):
        s_b, d_b, w_b, sem = slots[k]
        off = b * E + c * AGG_CH
        pltpu.async_copy(src_hbm.at[pl.ds(off, AGG_CH)], s_b, sem)
        pltpu.async_copy(dst_hbm.at[pl.ds(off, AGG_CH)], d_b, sem)
        pltpu.async_copy(ew_hbm.at[pl.ds(off, AGG_CH)], w_b, sem)

    def drain(k):
        s_b, d_b, w_b, sem = slots[k]
        pltpu.make_async_copy(src_hbm.at[pl.ds(0, AGG_CH)], s_b, sem).wait()
        pltpu.make_async_copy(dst_hbm.at[pl.ds(0, AGG_CH)], d_b, sem).wait()
        pltpu.make_async_copy(ew_hbm.at[pl.ds(0, AGG_CH)], w_b, sem).wait()

    issue(0, 0)
    issue(1, 1)
    pltpu.sync_copy(tab_hbm.at[pl.ds(wid * N, N)], tab_v)
    _zero_fill(acc_v)

    def pair(g, _):
        for k in range(2):
            c = g * 2 + k
            s_b, d_b, w_b, _sem = slots[k]
            drain(k)

            @plsc.parallel_loop(0, AGG_CH // L, unroll=8)
            def vec(i, s_b=s_b, d_b=d_b, w_b=w_b):
                o = i * L
                g_v = plsc.load_gather(tab_v, [s_b[pl.ds(o, L)]])
                m = g_v * w_b[pl.ds(o, L)]
                plsc.addupdate_scatter(acc_v, [d_b[pl.ds(o, L)]], m)

            @pl.when(c + 2 < n_chunks)
            def _(c=c, k=k):
                issue(c + 2, k)
        return 0

    lax.fori_loop(0, n_chunks // 2, pair, 0)
    pltpu.sync_copy(acc_v, agg_hbm.at[pl.ds(wid * N, N)])


def _tc_prep_body(cs_ref, cd_ref, ft_ref, xt_ref, rso_ref, rsi_ref):
    deg_o = jnp.sum(cs_ref[...], axis=0)
    deg_i = jnp.sum(cd_ref[...], axis=0)
    rs_o = lax.rsqrt(jnp.maximum(deg_o, 1.0))
    rs_i = lax.rsqrt(jnp.maximum(deg_i, 1.0))
    ft_t = jnp.transpose(ft_ref[...][0])
    xt_ref[...] = ft_t[None] * rs_o[None, None, :]
    rso_ref[...] = rs_o[None, None, :]
    rsi_ref[...] = rs_i[None, None, :]


def _tc_prep(cnt_src, cnt_dst, feat_t):
    return pl.pallas_call(
        _tc_prep_body,
        grid=(B,),
        compiler_params=pltpu.CompilerParams(
            vmem_limit_bytes=100 * 1024 * 1024),
        in_specs=[
            pl.BlockSpec((8, N), lambda b: (b, 0)),
            pl.BlockSpec((8, N), lambda b: (b, 0)),
            pl.BlockSpec((1, N, D_IN), lambda b: (b, 0, 0)),
        ],
        out_specs=[
            pl.BlockSpec((1, D_IN, N), lambda b: (b, 0, 0)),
            pl.BlockSpec((1, 1, N), lambda b: (b, 0, 0)),
            pl.BlockSpec((1, 1, N), lambda b: (b, 0, 0)),
        ],
        out_shape=[
            jax.ShapeDtypeStruct((B, D_IN, N), jnp.float32),
            jax.ShapeDtypeStruct((B, 1, N), jnp.float32),
            jax.ShapeDtypeStruct((B, 1, N), jnp.float32),
        ],
    )(cnt_src, cnt_dst, feat_t)


def _tc_mid_body(agg_ref, rsi_ref, rso_ref, w1t_ref, b1_ref, w2t_ref, out_ref):
    t = agg_ref[...][0] * rsi_ref[...][0]
    h = jnp.dot(w1t_ref[...], t, preferred_element_type=jnp.float32,
                precision=lax.Precision.HIGHEST) + b1_ref[...]
    h = jnp.where(h >= 0.0, h, 0.01 * h)
    g2 = jnp.dot(w2t_ref[...], h * rso_ref[...][0],
                 preferred_element_type=jnp.float32,
                 precision=lax.Precision.HIGHEST)
    out_ref[...] = g2[None]


def _tc_mid(agg1, rs_i, rs_o, w1t, b1c, w2t):
    return pl.pallas_call(
        _tc_mid_body,
        grid=(B,),
        in_specs=[
            pl.BlockSpec((1, D_IN, N), lambda b: (b, 0, 0)),
            pl.BlockSpec((1, 1, N), lambda b: (b, 0, 0)),
            pl.BlockSpec((1, 1, N), lambda b: (b, 0, 0)),
            pl.BlockSpec((D_HID, D_IN), lambda b: (0, 0)),
            pl.BlockSpec((D_HID, 1), lambda b: (0, 0)),
            pl.BlockSpec((D_OUT, D_HID), lambda b: (0, 0)),
        ],
        out_specs=pl.BlockSpec((1, D_OUT, N), lambda b: (b, 0, 0)),
        out_shape=jax.ShapeDtypeStruct((B, D_OUT, N), jnp.float32),
    )(agg1, rs_i, rs_o, w1t, b1c, w2t)


def _tc_final_body(agg_ref, rsi_ref, b2_ref, out_ref):
    o = agg_ref[...][0] * rsi_ref[...][0] + b2_ref[...]
    out_ref[...] = jnp.transpose(o[:, :CBG])[None]


def _tc_final(agg2, rs_i, b2c):
    return pl.pallas_call(
        _tc_final_body,
        grid=(B,),
        compiler_params=pltpu.CompilerParams(
            vmem_limit_bytes=100 * 1024 * 1024),
        in_specs=[
            pl.BlockSpec((1, D_OUT, N), lambda b: (b, 0, 0)),
            pl.BlockSpec((1, 1, N), lambda b: (b, 0, 0)),
            pl.BlockSpec((D_OUT, 1), lambda b: (0, 0)),
        ],
        out_specs=pl.BlockSpec((1, CBG, D_OUT), lambda b: (b, 0, 0)),
        out_shape=jax.ShapeDtypeStruct((B, CBG, D_OUT), jnp.float32),
    )(agg2, rs_i, b2c)


def kernel(edge_index, edge_weight, cbg_encode, poi_encode, W1, b1, W2, b2):
    feat = jnp.concatenate((cbg_encode, poi_encode), axis=1)
    src_flat = edge_index[:, 0, :].reshape(-1)
    dst_flat = edge_index[:, 1, :].reshape(-1)
    ew_flat = edge_weight.reshape(-1)

    cnt_src, cnt_dst = _sc_degrees(src_flat, dst_flat)
    x_t, rs_o, rs_i = _tc_prep(cnt_src.reshape(NW, N),
                               cnt_dst.reshape(NW, N), feat)

    agg1 = _sc_aggregate(src_flat, dst_flat, ew_flat,
                         x_t.reshape(NW * N))
    g2_t = _tc_mid(agg1.reshape(B, D_IN, N), rs_i, rs_o,
                   W1.T, b1.reshape(D_HID, 1), W2.T)

    agg2 = _sc_aggregate(src_flat, dst_flat, ew_flat,
                         g2_t.reshape(NW * N))
    return _tc_final(agg2.reshape(B, D_OUT, N), rs_i,
                     b2.reshape(D_OUT, 1))


# speedup vs baseline: 1.0952x; 1.0952x over previous
import functools

import jax
import jax.numpy as jnp
from jax import lax
from jax.experimental import pallas as pl
from jax.experimental.pallas import tpu as pltpu
from jax.experimental.pallas import tpu_sc as plsc

B = 4
N = 50000
E = 1600000
CBG = 30000
D_IN = 8
D_HID = 16
D_OUT = 8

NC, NS, L = 2, 16, 16
NW = NC * NS
N_VECS = N // L

DEG_CH = 4000
DEG_EDGES = E // 8
AGG_CH = 4000

_SC_MESH = plsc.VectorSubcoreMesh(
    core_axis_name="c", subcore_axis_name="s", num_cores=NC, num_subcores=NS)
_SC_PARAMS = pltpu.CompilerParams(needs_layout_passes=False)


def _zero_fill(ref):
    zeros = jnp.zeros((L,), jnp.float32)

    @plsc.parallel_loop(0, N_VECS, unroll=8)
    def body(i):
        ref[pl.ds(i * L, L)] = zeros


@functools.partial(
    pl.kernel, mesh=_SC_MESH, compiler_params=_SC_PARAMS,
    out_type=(jax.ShapeDtypeStruct((NW * N,), jnp.float32),
              jax.ShapeDtypeStruct((NW * N,), jnp.float32)),
    scratch_types=[
        pltpu.VMEM((N,), jnp.float32),
        pltpu.VMEM((N,), jnp.float32),
        pltpu.VMEM((DEG_CH,), jnp.int32),
        pltpu.VMEM((DEG_CH,), jnp.int32),
        pltpu.VMEM((DEG_CH,), jnp.int32),
        pltpu.VMEM((DEG_CH,), jnp.int32),
        pltpu.SemaphoreType.DMA,
        pltpu.SemaphoreType.DMA,
    ],
)
def _sc_degrees(src_hbm, dst_hbm, cnt_src_hbm, cnt_dst_hbm,
                cs_v, cd_v, src_b0, src_b1, dst_b0, dst_b1, sem0, sem1):
    wid = lax.axis_index("s") * NC + lax.axis_index("c")
    b = wid // 8
    base = b * E + (wid % 8) * DEG_EDGES
    n_chunks = DEG_EDGES // DEG_CH
    slots = ((src_b0, dst_b0, sem0), (src_b1, dst_b1, sem1))

    def issue(c, k):
        s_b, d_b, sem = slots[k]
        off = base + c * DEG_CH
        pltpu.async_copy(src_hbm.at[pl.ds(off, DEG_CH)], s_b, sem)
        pltpu.async_copy(dst_hbm.at[pl.ds(off, DEG_CH)], d_b, sem)

    def drain(k):
        s_b, d_b, sem = slots[k]
        pltpu.make_async_copy(src_hbm.at[pl.ds(0, DEG_CH)], s_b, sem).wait()
        pltpu.make_async_copy(dst_hbm.at[pl.ds(0, DEG_CH)], d_b, sem).wait()

    issue(0, 0)
    issue(1, 1)
    _zero_fill(cs_v)
    _zero_fill(cd_v)
    ones = jnp.ones((L,), jnp.float32)

    def pair(g, _):
        for k in range(2):
            c = g * 2 + k
            s_b, d_b, _sem = slots[k]
            drain(k)

            @plsc.parallel_loop(0, DEG_CH // L, unroll=8)
            def vec(i, s_b=s_b, d_b=d_b):
                o = i * L
                plsc.addupdate_scatter(cs_v, [s_b[pl.ds(o, L)]], ones)
                plsc.addupdate_scatter(cd_v, [d_b[pl.ds(o, L)]], ones)

            @pl.when(c + 2 < n_chunks)
            def _(c=c, k=k):
                issue(c + 2, k)
        return 0

    lax.fori_loop(0, n_chunks // 2, pair, 0)
    pltpu.sync_copy(cs_v, cnt_src_hbm.at[pl.ds(wid * N, N)])
    pltpu.sync_copy(cd_v, cnt_dst_hbm.at[pl.ds(wid * N, N)])


@functools.partial(
    pl.kernel, mesh=_SC_MESH, compiler_params=_SC_PARAMS,
    out_type=jax.ShapeDtypeStruct((NW * N,), jnp.float32),
    scratch_types=[
        pltpu.VMEM((N,), jnp.float32),
        pltpu.VMEM((N,), jnp.float32),
        pltpu.VMEM((AGG_CH,), jnp.int32),
        pltpu.VMEM((AGG_CH,), jnp.int32),
        pltpu.VMEM((AGG_CH,), jnp.int32),
        pltpu.VMEM((AGG_CH,), jnp.int32),
        pltpu.VMEM((AGG_CH,), jnp.float32),
        pltpu.VMEM((AGG_CH,), jnp.float32),
        pltpu.SemaphoreType.DMA,
        pltpu.SemaphoreType.DMA,
    ],
)
def _sc_aggregate(src_hbm, dst_hbm, ew_hbm, tab_hbm, agg_hbm,
                  tab_v, acc_v, src_b0, src_b1, dst_b0, dst_b1,
                  ew_b0, ew_b1, sem0, sem1):
    wid = lax.axis_index("s") * NC + lax.axis_index("c")
    b = wid // 8
    n_chunks = E // AGG_CH
    slots = ((src_b0, dst_b0, ew_b0, sem0), (src_b1, dst_b1, ew_b1, sem1))

    def issue(c, k):
        s_b, d_b, w_b, sem = slots[k]
        off = b * E + c * AGG_CH
        pltpu.async_copy(src_hbm.at[pl.ds(off, AGG_CH)], s_b, sem)
        pltpu.async_copy(dst_hbm.at[pl.ds(off, AGG_CH)], d_b, sem)
        pltpu.async_copy(ew_hbm.at[pl.ds(off, AGG_CH)], w_b, sem)

    def drain(k):
        s_b, d_b, w_b, sem = slots[k]
        pltpu.make_async_copy(src_hbm.at[pl.ds(0, AGG_CH)], s_b, sem).wait()
        pltpu.make_async_copy(dst_hbm.at[pl.ds(0, AGG_CH)], d_b, sem).wait()
        pltpu.make_async_copy(ew_hbm.at[pl.ds(0, AGG_CH)], w_b, sem).wait()

    issue(0, 0)
    issue(1, 1)
    pltpu.sync_copy(tab_hbm.at[pl.ds(wid * N, N)], tab_v)
    _zero_fill(acc_v)

    def pair(g, _):
        for k in range(2):
            c = g * 2 + k
            s_b, d_b, w_b, _sem = slots[k]
            drain(k)

            @plsc.parallel_loop(0, AGG_CH // L, unroll=8)
            def vec(i, s_b=s_b, d_b=d_b, w_b=w_b):
                o = i * L
                g_v = plsc.load_gather(tab_v, [s_b[pl.ds(o, L)]])
                m = g_v * w_b[pl.ds(o, L)]
                plsc.addupdate_scatter(acc_v, [d_b[pl.ds(o, L)]], m)

            @pl.when(c + 2 < n_chunks)
            def _(c=c, k=k):
                issue(c + 2, k)
        return 0

    lax.fori_loop(0, n_chunks // 2, pair, 0)
    pltpu.sync_copy(acc_v, agg_hbm.at[pl.ds(wid * N, N)])


def _tc_prep_body(cs_ref, cd_ref, ft_ref, xt_ref, rso_ref, rsi_ref):
    deg_o = jnp.sum(cs_ref[...], axis=0)
    deg_i = jnp.sum(cd_ref[...], axis=0)
    rs_o = lax.rsqrt(jnp.maximum(deg_o, 1.0))
    rs_i = lax.rsqrt(jnp.maximum(deg_i, 1.0))
    xt_ref[...] = ft_ref[...] * rs_o[None, None, :]
    rso_ref[...] = rs_o[None, None, :]
    rsi_ref[...] = rs_i[None, None, :]


def _tc_prep(cnt_src, cnt_dst, feat_t):
    return pl.pallas_call(
        _tc_prep_body,
        grid=(B,),
        compiler_params=pltpu.CompilerParams(
            vmem_limit_bytes=100 * 1024 * 1024),
        in_specs=[
            pl.BlockSpec((8, N), lambda b: (b, 0)),
            pl.BlockSpec((8, N), lambda b: (b, 0)),
            pl.BlockSpec((1, D_IN, N), lambda b: (b, 0, 0)),
        ],
        out_specs=[
            pl.BlockSpec((1, D_IN, N), lambda b: (b, 0, 0)),
            pl.BlockSpec((1, 1, N), lambda b: (b, 0, 0)),
            pl.BlockSpec((1, 1, N), lambda b: (b, 0, 0)),
        ],
        out_shape=[
            jax.ShapeDtypeStruct((B, D_IN, N), jnp.float32),
            jax.ShapeDtypeStruct((B, 1, N), jnp.float32),
            jax.ShapeDtypeStruct((B, 1, N), jnp.float32),
        ],
    )(cnt_src, cnt_dst, feat_t)


def _tc_mid_body(agg_ref, rsi_ref, rso_ref, w1t_ref, b1_ref, w2t_ref, out_ref):
    t = agg_ref[...][0] * rsi_ref[...][0]
    h = jnp.dot(w1t_ref[...], t, preferred_element_type=jnp.float32,
                precision=lax.Precision.HIGHEST) + b1_ref[...]
    h = jnp.where(h >= 0.0, h, 0.01 * h)
    g2 = jnp.dot(w2t_ref[...], h * rso_ref[...][0],
                 preferred_element_type=jnp.float32,
                 precision=lax.Precision.HIGHEST)
    out_ref[...] = g2[None]


def _tc_mid(agg1, rs_i, rs_o, w1t, b1c, w2t):
    return pl.pallas_call(
        _tc_mid_body,
        grid=(B,),
        in_specs=[
            pl.BlockSpec((1, D_IN, N), lambda b: (b, 0, 0)),
            pl.BlockSpec((1, 1, N), lambda b: (b, 0, 0)),
            pl.BlockSpec((1, 1, N), lambda b: (b, 0, 0)),
            pl.BlockSpec((D_HID, D_IN), lambda b: (0, 0)),
            pl.BlockSpec((D_HID, 1), lambda b: (0, 0)),
            pl.BlockSpec((D_OUT, D_HID), lambda b: (0, 0)),
        ],
        out_specs=pl.BlockSpec((1, D_OUT, N), lambda b: (b, 0, 0)),
        out_shape=jax.ShapeDtypeStruct((B, D_OUT, N), jnp.float32),
    )(agg1, rs_i, rs_o, w1t, b1c, w2t)


def _tc_final_body(agg_ref, rsi_ref, b2_ref, out_ref):
    o = agg_ref[...][0] * rsi_ref[...][0] + b2_ref[...]
    out_ref[...] = o[:, :CBG][None]


def _tc_final(agg2, rs_i, b2c):
    return pl.pallas_call(
        _tc_final_body,
        grid=(B,),
        compiler_params=pltpu.CompilerParams(
            vmem_limit_bytes=100 * 1024 * 1024),
        in_specs=[
            pl.BlockSpec((1, D_OUT, N), lambda b: (b, 0, 0)),
            pl.BlockSpec((1, 1, N), lambda b: (b, 0, 0)),
            pl.BlockSpec((D_OUT, 1), lambda b: (0, 0)),
        ],
        out_specs=pl.BlockSpec((1, D_OUT, CBG), lambda b: (b, 0, 0)),
        out_shape=jax.ShapeDtypeStruct((B, D_OUT, CBG), jnp.float32),
    )(agg2, rs_i, b2c)


def kernel(edge_index, edge_weight, cbg_encode, poi_encode, W1, b1, W2, b2):
    feat_t = jnp.concatenate(
        (cbg_encode, poi_encode), axis=1).transpose(0, 2, 1)
    src_flat = edge_index[:, 0, :].reshape(-1)
    dst_flat = edge_index[:, 1, :].reshape(-1)
    ew_flat = edge_weight.reshape(-1)

    cnt_src, cnt_dst = _sc_degrees(src_flat, dst_flat)
    x_t, rs_o, rs_i = _tc_prep(cnt_src.reshape(NW, N),
                               cnt_dst.reshape(NW, N), feat_t)

    agg1 = _sc_aggregate(src_flat, dst_flat, ew_flat,
                         x_t.reshape(NW * N))
    g2_t = _tc_mid(agg1.reshape(B, D_IN, N), rs_i, rs_o,
                   W1.T, b1.reshape(D_HID, 1), W2.T)

    agg2 = _sc_aggregate(src_flat, dst_flat, ew_flat,
                         g2_t.reshape(NW * N))
    out_t = _tc_final(agg2.reshape(B, D_OUT, N), rs_i,
                      b2.reshape(D_OUT, 1))
    return out_t.transpose(0, 2, 1)
